# baseline (speedup 1.0000x reference)
import functools

import jax
import jax.numpy as jnp
from jax import lax
from jax.experimental import pallas as pl
from jax.experimental.pallas import tpu as pltpu
from jax.experimental.pallas import tpu_sc as plsc

NENTITY = 1000000
NRELATION = 1000
HIDDEN = 128
GAMMA = 12.0
BATCH = 4096

NUM_CORES = 2
NUM_SUBCORES = 16
LANES = 16
NUM_WORKERS = NUM_CORES * NUM_SUBCORES
BPW = BATCH // NUM_WORKERS
BLK0 = 48
BLK1 = BPW - BLK0
PITCH = 17

_mesh = plsc.VectorSubcoreMesh(core_axis_name="c", subcore_axis_name="s")


@functools.partial(
    pl.kernel,
    mesh=_mesh,
    compiler_params=pltpu.CompilerParams(needs_layout_passes=False),
    out_type=jax.ShapeDtypeStruct((BATCH,), jnp.float32),
    scratch_types=[
        pltpu.VMEM((BPW,), jnp.int32),
        pltpu.VMEM((BPW,), jnp.int32),
        pltpu.VMEM((BPW,), jnp.int32),
        pltpu.VMEM((BPW, HIDDEN), jnp.float32),
        pltpu.VMEM((BPW, HIDDEN), jnp.float32),
        pltpu.VMEM((BPW, HIDDEN), jnp.float32),
        pltpu.VMEM((BPW * PITCH,), jnp.float32),
        pltpu.VMEM((BPW,), jnp.float32),
        pltpu.SemaphoreType.DMA,
        pltpu.SemaphoreType.DMA,
        pltpu.SemaphoreType.DMA,
        pltpu.SemaphoreType.DMA,
        pltpu.SemaphoreType.DMA,
    ],
)
def _transe_sc(hidx_hbm, ridx_hbm, tidx_hbm, ent_hbm, rel_hbm, out_hbm,
               idx_h, idx_r, idx_t, rows_h, rows_r, rows_t, acc_buf, out_v,
               sem_b0, sem_b1, sem_ih, sem_ir, sem_it):
    wid = lax.axis_index("s") * NUM_CORES + lax.axis_index("c")
    base = wid * BPW

    cih = pltpu.async_copy(hidx_hbm.at[pl.ds(base, BPW)], idx_h, sem_ih)
    cir = pltpu.async_copy(ridx_hbm.at[pl.ds(base, BPW)], idx_r, sem_ir)
    cit = pltpu.async_copy(tidx_hbm.at[pl.ds(base, BPW)], idx_t, sem_it)

    def fire(lo, n, sem):
        s = pl.ds(lo, n)
        return (
            pltpu.async_copy(ent_hbm.at[idx_h.at[s]], rows_h.at[s], sem),
            pltpu.async_copy(rel_hbm.at[idx_r.at[s]], rows_r.at[s], sem),
            pltpu.async_copy(ent_hbm.at[idx_t.at[s]], rows_t.at[s], sem),
        )

    def sample_body(i, carry):
        acc = jnp.zeros((LANES,), jnp.float32)
        for j in range(HIDDEN // LANES):
            h = rows_h[i, pl.ds(j * LANES, LANES)]
            r = rows_r[i, pl.ds(j * LANES, LANES)]
            t = rows_t[i, pl.ds(j * LANES, LANES)]
            acc = acc + jnp.abs(h + r - t)
        acc_buf[pl.ds(i * PITCH, LANES)] = acc
        return carry

    s0 = pl.ds(0, BLK0)
    cih.wait()
    c0h = pltpu.async_copy(ent_hbm.at[idx_h.at[s0]], rows_h.at[s0], sem_b0)
    cir.wait()
    c0r = pltpu.async_copy(rel_hbm.at[idx_r.at[s0]], rows_r.at[s0], sem_b0)
    cit.wait()
    c0t = pltpu.async_copy(ent_hbm.at[idx_t.at[s0]], rows_t.at[s0], sem_b0)

    c0h.wait()
    c0r.wait()
    c0t.wait()
    pending = fire(BLK0, BLK1, sem_b1)
    lax.fori_loop(0, BLK0, sample_body, None, unroll=2)
    for c in pending:
        c.wait()
    lax.fori_loop(BLK0, BPW, sample_body, None, unroll=2)

    lane_pitch = lax.iota(jnp.int32, LANES) * PITCH

    def group_body(g, carry):
        tot = jnp.zeros((LANES,), jnp.float32)
        for k in range(LANES):
            tot = tot + plsc.load_gather(
                acc_buf, [lane_pitch + (g * (LANES * PITCH) + k)])
        out_v[pl.ds(pl.multiple_of(g * LANES, LANES), LANES)] = GAMMA - tot
        return carry

    lax.fori_loop(0, BPW // LANES, group_body, None)

    pltpu.sync_copy(out_v, out_hbm.at[pl.ds(base, BPW)])


def kernel(sample, entity_embedding, relation_embedding):
    head_idx = sample[:, 0]
    rel_idx = sample[:, 1]
    tail_idx = sample[:, 2]
    score = _transe_sc(head_idx, rel_idx, tail_idx,
                       entity_embedding, relation_embedding)
    return score.reshape(BATCH, 1)


# speedup vs baseline: 1.0106x; 1.0106x over previous
import functools

import jax
import jax.numpy as jnp
from jax import lax
from jax.experimental import pallas as pl
from jax.experimental.pallas import tpu as pltpu
from jax.experimental.pallas import tpu_sc as plsc

NENTITY = 1000000
NRELATION = 1000
HIDDEN = 128
GAMMA = 12.0
BATCH = 4096

NUM_CORES = 2
NUM_SUBCORES = 16
LANES = 16
NUM_WORKERS = NUM_CORES * NUM_SUBCORES
BPW = BATCH // NUM_WORKERS
NBLK = 2
BLK = BPW // NBLK
PITCH = 17

_mesh = plsc.VectorSubcoreMesh(core_axis_name="c", subcore_axis_name="s")


@functools.partial(
    pl.kernel,
    mesh=_mesh,
    compiler_params=pltpu.CompilerParams(needs_layout_passes=False),
    out_type=jax.ShapeDtypeStruct((BATCH,), jnp.float32),
    scratch_types=[
        pltpu.VMEM((BPW,), jnp.int32),
        pltpu.VMEM((BPW,), jnp.int32),
        pltpu.VMEM((BPW,), jnp.int32),
        pltpu.VMEM((BPW, HIDDEN), jnp.float32),
        pltpu.VMEM((BPW, HIDDEN), jnp.float32),
        pltpu.VMEM((BPW, HIDDEN), jnp.float32),
        pltpu.VMEM((BPW * PITCH,), jnp.float32),
        pltpu.VMEM((BPW,), jnp.float32),
        pltpu.SemaphoreType.DMA,
        pltpu.SemaphoreType.DMA,
        pltpu.SemaphoreType.DMA,
        pltpu.SemaphoreType.DMA,
        pltpu.SemaphoreType.DMA,
    ],
)
def _transe_sc(hidx_hbm, ridx_hbm, tidx_hbm, ent_hbm, rel_hbm, out_hbm,
               idx_h, idx_r, idx_t, rows_h, rows_r, rows_t, acc_buf, out_v,
               sem_b0, sem_b1, sem_ih, sem_ir, sem_it):
    wid = lax.axis_index("s") * NUM_CORES + lax.axis_index("c")
    base = wid * BPW

    cih = pltpu.async_copy(hidx_hbm.at[pl.ds(base, BPW)], idx_h, sem_ih)
    cir = pltpu.async_copy(ridx_hbm.at[pl.ds(base, BPW)], idx_r, sem_ir)
    cit = pltpu.async_copy(tidx_hbm.at[pl.ds(base, BPW)], idx_t, sem_it)

    sems = (sem_b0, sem_b1)

    def fire(b):
        s = pl.ds(b * BLK, BLK)
        return (
            pltpu.async_copy(ent_hbm.at[idx_h.at[s]], rows_h.at[s], sems[b]),
            pltpu.async_copy(rel_hbm.at[idx_r.at[s]], rows_r.at[s], sems[b]),
            pltpu.async_copy(ent_hbm.at[idx_t.at[s]], rows_t.at[s], sems[b]),
        )

    def sample_body(i, carry):
        acc = jnp.zeros((LANES,), jnp.float32)
        for j in range(HIDDEN // LANES):
            h = rows_h[i, pl.ds(j * LANES, LANES)]
            r = rows_r[i, pl.ds(j * LANES, LANES)]
            t = rows_t[i, pl.ds(j * LANES, LANES)]
            acc = acc + jnp.abs(h + r - t)
        acc_buf[pl.ds(i * PITCH, LANES)] = acc
        return carry

    s0 = pl.ds(0, BLK)
    cih.wait()
    c0h = pltpu.async_copy(ent_hbm.at[idx_h.at[s0]], rows_h.at[s0], sem_b0)
    cir.wait()
    c0r = pltpu.async_copy(rel_hbm.at[idx_r.at[s0]], rows_r.at[s0], sem_b0)
    cit.wait()
    c0t = pltpu.async_copy(ent_hbm.at[idx_t.at[s0]], rows_t.at[s0], sem_b0)

    pending = (c0h, c0r, c0t)
    for b in range(NBLK):
        for c in pending:
            c.wait()
        if b + 1 < NBLK:
            pending = fire(b + 1)
        lax.fori_loop(b * BLK, (b + 1) * BLK, sample_body, None, unroll=2)

    lane_pitch = lax.iota(jnp.int32, LANES) * PITCH

    def group_body(g, carry):
        tot = jnp.zeros((LANES,), jnp.float32)
        for k in range(LANES):
            tot = tot + plsc.load_gather(
                acc_buf, [lane_pitch + (g * (LANES * PITCH) + k)])
        out_v[pl.ds(pl.multiple_of(g * LANES, LANES), LANES)] = GAMMA - tot
        return carry

    lax.fori_loop(0, BPW // LANES, group_body, None)

    pltpu.sync_copy(out_v, out_hbm.at[pl.ds(base, BPW)])


def kernel(sample, entity_embedding, relation_embedding):
    head_idx = sample[:, 0]
    rel_idx = sample[:, 1]
    tail_idx = sample[:, 2]
    score = _transe_sc(head_idx, rel_idx, tail_idx,
                       entity_embedding, relation_embedding)
    return score.reshape(BATCH, 1)
